# baseline (speedup 1.0000x reference)
import jax
import jax.numpy as jnp
from jax import lax
from jax.experimental import pallas as pl
from jax.experimental.pallas import tpu as pltpu
from jax.experimental.pallas import tpu_sc as plsc

D = 128
LANES = 16
NUM_WORKERS = 32
CHUNK = 128
IDXW = 128
HALVES = CHUNK // IDXW
ROW_UNROLL = 4


def _sc_body(xs_hbm, e0_hbm, e1_hbm, e2_hbm, out_hbm,
             idxA, idxB, bufA0, bufA1, bufA2, bufB0, bufB1, bufB2,
             isemA, isemB, gsemA, gsemB, osemA, osemB):
    n_tokens = out_hbm.shape[0]
    per_worker = n_tokens // NUM_WORKERS
    n_chunks = per_worker // CHUNK
    wid = lax.axis_index("s") * 2 + lax.axis_index("c")
    base = wid * per_worker
    idx_base = wid * n_chunks

    sets = (
        (idxA, (bufA0, bufA1, bufA2), isemA, gsemA, osemA),
        (idxB, (bufB0, bufB1, bufB2), isemB, gsemB, osemB),
    )

    def idx_desc(g, s):
        idx, _, isem, _, _ = s
        return pltpu.make_async_copy(xs_hbm.at[idx_base + g], idx, isem)

    def out_desc(g, s):
        _, bufs, _, _, osem = s
        return pltpu.make_async_copy(
            bufs[0], out_hbm.at[pl.ds(base + g * CHUNK, CHUNK)], osem)

    def gather_descs(s):
        idx, bufs, _, gsem, _ = s
        tables = (e0_hbm, e1_hbm, e2_hbm)
        descs = []
        for st in range(3):
            for h in range(HALVES):
                descs.append(pltpu.make_async_copy(
                    tables[st].at[idx.at[st * HALVES + h]],
                    bufs[st].at[pl.ds(h * IDXW, IDXW)],
                    gsem))
        return tuple(descs)

    def fire_gathers(s):
        for c in gather_descs(s):
            c.start()

    def wait_gathers(s):
        for c in gather_descs(s):
            c.wait()

    d = idx_desc(0, sets[0])
    d.start()
    d.wait()
    fire_gathers(sets[0])
    idx_desc(min(1, n_chunks - 1), sets[1]).start()

    def pair_body(gg, carry):
        for b in range(2):
            g = gg * 2 + b
            cur = sets[b]
            nxt = sets[1 - b]
            _, bufs, _, _, _ = cur
            wait_gathers(cur)
            idx_desc(g, nxt).wait()

            @pl.when(g > 0)
            def _():
                out_desc(g, nxt).wait()

            fire_gathers(nxt)
            idx_desc(jnp.minimum(g + 2, n_chunks - 1), cur).start()

            def add_rows(i, c):
                b0, b1, b2 = bufs
                for r in range(ROW_UNROLL):
                    row = i * ROW_UNROLL + r
                    for j in range(D // LANES):
                        sl = pl.ds(j * LANES, LANES)
                        plsc.addupdate(b0.at[row, sl], b1[row, sl] + b2[row, sl])
                return c

            lax.fori_loop(0, CHUNK // ROW_UNROLL, add_rows, 0)
            out_desc(g, cur).start()
        return carry

    lax.fori_loop(0, n_chunks // 2, pair_body, 0)

    wait_gathers(sets[0])
    idx_desc(0, sets[1]).wait()
    out_desc(n_chunks - 1, sets[1]).wait()


def _make_kernel(n_tokens):
    mesh = plsc.VectorSubcoreMesh(core_axis_name="c", subcore_axis_name="s")
    return pl.kernel(
        _sc_body,
        out_type=jax.ShapeDtypeStruct((n_tokens, D), jnp.float32),
        mesh=mesh,
        scratch_types=[
            pltpu.VMEM((3 * HALVES, IDXW), jnp.int32),
            pltpu.VMEM((3 * HALVES, IDXW), jnp.int32),
            pltpu.VMEM((CHUNK, D), jnp.float32),
            pltpu.VMEM((CHUNK, D), jnp.float32),
            pltpu.VMEM((CHUNK, D), jnp.float32),
            pltpu.VMEM((CHUNK, D), jnp.float32),
            pltpu.VMEM((CHUNK, D), jnp.float32),
            pltpu.VMEM((CHUNK, D), jnp.float32),
            pltpu.SemaphoreType.DMA,
            pltpu.SemaphoreType.DMA,
            pltpu.SemaphoreType.DMA,
            pltpu.SemaphoreType.DMA,
            pltpu.SemaphoreType.DMA,
            pltpu.SemaphoreType.DMA,
        ],
    )


@jax.jit
def kernel(x, e0, e1, e2):
    b, t, _ = x.shape
    n_tokens = b * t
    per_worker = n_tokens // NUM_WORKERS
    n_chunks = per_worker // CHUNK
    xs = (x.astype(jnp.int32)
          .reshape(NUM_WORKERS, n_chunks, CHUNK, 3)
          .transpose(0, 1, 3, 2)
          .reshape(NUM_WORKERS * n_chunks, 3 * HALVES, IDXW))
    out = _make_kernel(n_tokens)(xs, e0, e1, e2)
    return out.reshape(b, t, D)


# speedup vs baseline: 1.0344x; 1.0344x over previous
import jax
import jax.numpy as jnp
from jax import lax
from jax.experimental import pallas as pl
from jax.experimental.pallas import tpu as pltpu
from jax.experimental.pallas import tpu_sc as plsc

D = 128
LANES = 16
NUM_WORKERS = 32
CHUNK = 64
SETS = 4
ROW_UNROLL = 8


def _sc_body(xs_hbm, e0_hbm, e1_hbm, e2_hbm, out_hbm, *scratch):
    idxs = scratch[0:SETS]
    bufs = [scratch[SETS + 3 * s: SETS + 3 * s + 3] for s in range(SETS)]
    isems = scratch[SETS + 3 * SETS: 2 * SETS + 3 * SETS]
    gsems = scratch[2 * SETS + 3 * SETS: 3 * SETS + 3 * SETS]
    osems = scratch[3 * SETS + 3 * SETS: 4 * SETS + 3 * SETS]

    n_tokens = out_hbm.shape[0]
    per_worker = n_tokens // NUM_WORKERS
    n_chunks = per_worker // CHUNK
    wid = lax.axis_index("s") * 2 + lax.axis_index("c")
    base = wid * per_worker
    idx_base = wid * n_chunks

    def idx_desc(g, s):
        return pltpu.make_async_copy(xs_hbm.at[idx_base + g], idxs[s], isems[s])

    def out_desc(g, s):
        return pltpu.make_async_copy(
            bufs[s][0], out_hbm.at[pl.ds(base + g * CHUNK, CHUNK)], osems[s])

    def gather_descs(s):
        return (pltpu.make_async_copy(e0_hbm.at[idxs[s].at[0]], bufs[s][0], gsems[s]),
                pltpu.make_async_copy(e1_hbm.at[idxs[s].at[1]], bufs[s][1], gsems[s]),
                pltpu.make_async_copy(e2_hbm.at[idxs[s].at[2]], bufs[s][2], gsems[s]))

    def fire_gathers(s):
        for c in gather_descs(s):
            c.start()

    def wait_gathers(s):
        for c in gather_descs(s):
            c.wait()

    for s in range(SETS):
        idx_desc(s, s).start()
    idx_desc(0, 0).wait()
    fire_gathers(0)
    idx_desc(1, 1).wait()
    fire_gathers(1)

    def quad_body(gg, carry):
        for b in range(SETS):
            g = gg * SETS + b
            b0, b1, b2 = bufs[b]
            wait_gathers(b)

            @pl.when(g + SETS < n_chunks)
            def _():
                idx_desc(g + SETS, b).start()

            nxt = (b + 2) % SETS

            @pl.when(jnp.logical_and(g + 2 < n_chunks, g >= 2))
            def _():
                out_desc(g - 2, nxt).wait()

            @pl.when(g + 2 < n_chunks)
            def _():
                idx_desc(g + 2, nxt).wait()
                fire_gathers(nxt)

            def add_rows(i, c):
                for r in range(ROW_UNROLL):
                    row = i * ROW_UNROLL + r
                    for j in range(D // LANES):
                        sl = pl.ds(j * LANES, LANES)
                        plsc.addupdate(b0.at[row, sl], b1[row, sl] + b2[row, sl])
                return c

            lax.fori_loop(0, CHUNK // ROW_UNROLL, add_rows, 0)
            out_desc(g, b).start()
        return carry

    lax.fori_loop(0, n_chunks // SETS, quad_body, 0)

    for k in range(SETS):
        out_desc(n_chunks - SETS + k, (n_chunks - SETS + k) % SETS).wait()


def _make_kernel(n_tokens):
    mesh = plsc.VectorSubcoreMesh(core_axis_name="c", subcore_axis_name="s")
    scratch = (
        [pltpu.VMEM((3, CHUNK), jnp.int32) for _ in range(SETS)]
        + [pltpu.VMEM((CHUNK, D), jnp.float32) for _ in range(3 * SETS)]
        + [pltpu.SemaphoreType.DMA for _ in range(3 * SETS)]
    )
    return pl.kernel(
        _sc_body,
        out_type=jax.ShapeDtypeStruct((n_tokens, D), jnp.float32),
        mesh=mesh,
        scratch_types=scratch,
    )


@jax.jit
def kernel(x, e0, e1, e2):
    b, t, _ = x.shape
    n_tokens = b * t
    per_worker = n_tokens // NUM_WORKERS
    n_chunks = per_worker // CHUNK
    xs = (x.astype(jnp.int32)
          .reshape(NUM_WORKERS, n_chunks, CHUNK, 3)
          .transpose(0, 1, 3, 2)
          .reshape(NUM_WORKERS * n_chunks, 3, CHUNK))
    out = _make_kernel(n_tokens)(xs, e0, e1, e2)
    return out.reshape(b, t, D)
